# Initial kernel scaffold; baseline (speedup 1.0000x reference)
import jax
import jax.numpy as jnp
from jax.experimental import pallas as pl


def kernel(x, edge_index, eps, W1, b1, g1, be1, W2, b2, g2, be2):
    raise NotImplementedError("write your pallas kernel here")


# speedup vs baseline: 5.0524x; 5.0524x over previous
import functools

import jax
import jax.numpy as jnp
from jax import lax
from jax.experimental import pallas as pl
from jax.experimental.pallas import tpu as pltpu
from jax.experimental.pallas import tpu_sc as plsc

N = 10000
E = 160000
D = 256
HALF = 128
BN_EPS = 1e-5

NC = 2
NS = 16
EPT = E // NS
CHUNK = 80
NCH = EPT // CHUNK
NP = 10240
RPT = NP // NS

_sc_mesh = plsc.VectorSubcoreMesh(core_axis_name="c", subcore_axis_name="s")


@functools.partial(
    pl.kernel,
    mesh=_sc_mesh,
    out_type=[
        jax.ShapeDtypeStruct((NP, HALF), jnp.float32),
        jax.ShapeDtypeStruct((NP, HALF), jnp.float32),
    ],
    scratch_types=[
        pltpu.VMEM((NCH, CHUNK), jnp.int32),
        pltpu.VMEM((NCH, CHUNK), jnp.int32),
        pltpu.VMEM((CHUNK, HALF), jnp.float32),
        pltpu.VMEM_SHARED((NP, HALF), jnp.float32),
        pltpu.SemaphoreType.DMA,
    ],
)
def _sc_agg(xr_hbm, gidx_hbm, didx_hbm, zeros_hbm, out_lo, out_hi,
            gidx_v, didx_v, rows_v, agg_sh, sem):
    c = lax.axis_index("c")
    s = lax.axis_index("s")
    pltpu.sync_copy(gidx_hbm.at[c, s], gidx_v)
    pltpu.sync_copy(didx_hbm.at[s], didx_v)
    pltpu.sync_copy(zeros_hbm.at[pl.ds(s * RPT, RPT)],
                    agg_sh.at[pl.ds(s * RPT, RPT)])
    plsc.subcore_barrier()

    def body(g, carry):
        pltpu.async_copy(xr_hbm.at[gidx_v.at[g]], rows_v, sem).wait()
        pltpu.sync_copy(rows_v, agg_sh.at[didx_v.at[g]], add=True)
        return carry

    lax.fori_loop(0, NCH, body, 0)
    plsc.subcore_barrier()

    @pl.when(c == 0)
    def _():
        pltpu.sync_copy(agg_sh.at[pl.ds(s * RPT, RPT)],
                        out_lo.at[pl.ds(s * RPT, RPT)])

    @pl.when(c == 1)
    def _():
        pltpu.sync_copy(agg_sh.at[pl.ds(s * RPT, RPT)],
                        out_hi.at[pl.ds(s * RPT, RPT)])


RB = 2000


def _tc_mlp_body(s0_ref, x_ref, alo_ref, ahi_ref, w1_ref, b1_ref,
                 w2_ref, b2_ref, o_ref):
    x = x_ref[...]
    agg = jnp.concatenate([alo_ref[...], ahi_ref[...]], axis=1)
    h = s0_ref[...] * x + agg
    h = jnp.dot(h, w1_ref[...], preferred_element_type=jnp.float32)
    h = jnp.maximum(h + b1_ref[...], 0.0)
    h = jnp.dot(h, w2_ref[...], preferred_element_type=jnp.float32)
    h = jnp.maximum(h + b2_ref[...], 0.0)
    o_ref[...] = h + x


def kernel(x, edge_index, eps, W1, b1, g1, be1, W2, b2, g2, be2):
    src = edge_index[0]
    dst = edge_index[1]
    g_lo = src * 2
    gidx = jnp.stack([g_lo, g_lo + 1]).reshape(NC, NS, NCH, CHUNK)
    didx = dst.reshape(NS, NCH, CHUNK)
    xr = x.reshape(2 * N, HALF)
    zeros = jnp.zeros((NP, HALF), jnp.float32)

    agg_lo, agg_hi = _sc_agg(xr, gidx, didx, zeros)

    inv = 1.0 / jnp.sqrt(jnp.float32(1.0) + BN_EPS)
    s1 = g1 * inv
    W1f = W1 * s1[None, :]
    b1f = (b1 * s1 + be1)[None, :]
    s2 = g2 * inv
    W2f = W2 * s2[None, :]
    b2f = (b2 * s2 + be2)[None, :]
    s0 = jnp.broadcast_to(1.0 + eps, (1, D)).astype(jnp.float32)

    out = pl.pallas_call(
        _tc_mlp_body,
        grid=(N // RB,),
        in_specs=[
            pl.BlockSpec((1, D), lambda i: (0, 0)),
            pl.BlockSpec((RB, D), lambda i: (i, 0)),
            pl.BlockSpec((RB, HALF), lambda i: (i, 0)),
            pl.BlockSpec((RB, HALF), lambda i: (i, 0)),
            pl.BlockSpec((D, D), lambda i: (0, 0)),
            pl.BlockSpec((1, D), lambda i: (0, 0)),
            pl.BlockSpec((D, D), lambda i: (0, 0)),
            pl.BlockSpec((1, D), lambda i: (0, 0)),
        ],
        out_specs=pl.BlockSpec((RB, D), lambda i: (i, 0)),
        out_shape=jax.ShapeDtypeStruct((N, D), jnp.float32),
    )(s0, x, agg_lo, agg_hi, W1f, b1f, W2f, b2f)
    return out


# speedup vs baseline: 5.9699x; 1.1816x over previous
import functools

import jax
import jax.numpy as jnp
from jax import lax
from jax.experimental import pallas as pl
from jax.experimental.pallas import tpu as pltpu
from jax.experimental.pallas import tpu_sc as plsc

N = 10000
E = 160000
D = 256
HALF = 128
BN_EPS = 1e-5

NC = 2
NS = 16
EPT = E // NS
CHUNK = 40
NCH = EPT // CHUNK
NP = 10240
RPT = NP // NS
K = 5
NG = NCH // K

_sc_mesh = plsc.VectorSubcoreMesh(core_axis_name="c", subcore_axis_name="s")


@functools.partial(
    pl.kernel,
    mesh=_sc_mesh,
    out_type=[
        jax.ShapeDtypeStruct((NP, HALF), jnp.float32),
        jax.ShapeDtypeStruct((NP, HALF), jnp.float32),
    ],
    scratch_types=[
        pltpu.VMEM((2, K, CHUNK), jnp.int32),
        pltpu.VMEM((2, K, CHUNK), jnp.int32),
        pltpu.VMEM((K, CHUNK, HALF), jnp.float32),
        pltpu.VMEM_SHARED((NP, HALF), jnp.float32),
        pltpu.SemaphoreType.DMA,
        pltpu.SemaphoreType.DMA,
        pltpu.SemaphoreType.DMA,
        pltpu.SemaphoreType.DMA,
        pltpu.SemaphoreType.DMA,
        pltpu.SemaphoreType.DMA,
        pltpu.SemaphoreType.DMA,
    ],
)
def _sc_agg(xr_hbm, gidx_hbm, didx_hbm, zeros_hbm, out_lo, out_hi,
            gidx_v, didx_v, rows_v, agg_sh, s0, s1, s2, s3, s4, sig, sid):
    c = lax.axis_index("c")
    s = lax.axis_index("s")
    sems = (s0, s1, s2, s3, s4)
    pltpu.sync_copy(gidx_hbm.at[c, s, 0], gidx_v.at[0])
    pltpu.sync_copy(didx_hbm.at[s, 0], didx_v.at[0])
    pltpu.sync_copy(zeros_hbm.at[pl.ds(s * RPT, RPT)],
                    agg_sh.at[pl.ds(s * RPT, RPT)])
    plsc.subcore_barrier()

    def group(j, p):
        jn = jnp.minimum(j + 1, NG - 1)
        hg = pltpu.async_copy(gidx_hbm.at[c, s, jn], gidx_v.at[1 - p], sig)
        hd = pltpu.async_copy(didx_hbm.at[s, jn], didx_v.at[1 - p], sid)
        handles = []
        for k in range(K):
            handles.append(pltpu.async_copy(
                xr_hbm.at[gidx_v.at[p, k]], rows_v.at[k], sems[k]))
        for k in range(K):
            handles[k].wait()
            pltpu.sync_copy(rows_v.at[k], agg_sh.at[didx_v.at[p, k]],
                            add=True)
        hg.wait()
        hd.wait()

    def body(m, carry):
        group(2 * m, 0)
        group(2 * m + 1, 1)
        return carry

    lax.fori_loop(0, NG // 2, body, 0)
    plsc.subcore_barrier()

    @pl.when(c == 0)
    def _():
        pltpu.sync_copy(agg_sh.at[pl.ds(s * RPT, RPT)],
                        out_lo.at[pl.ds(s * RPT, RPT)])

    @pl.when(c == 1)
    def _():
        pltpu.sync_copy(agg_sh.at[pl.ds(s * RPT, RPT)],
                        out_hi.at[pl.ds(s * RPT, RPT)])


RB = 2000


def _tc_mlp_body(s0_ref, x_ref, alo_ref, ahi_ref, w1_ref, b1_ref,
                 w2_ref, b2_ref, o_ref):
    x = x_ref[...]
    agg = jnp.concatenate([alo_ref[...], ahi_ref[...]], axis=1)
    h = s0_ref[...] * x + agg
    h = jnp.dot(h, w1_ref[...], preferred_element_type=jnp.float32)
    h = jnp.maximum(h + b1_ref[...], 0.0)
    h = jnp.dot(h, w2_ref[...], preferred_element_type=jnp.float32)
    h = jnp.maximum(h + b2_ref[...], 0.0)
    o_ref[...] = h + x


def kernel(x, edge_index, eps, W1, b1, g1, be1, W2, b2, g2, be2):
    src = edge_index[0]
    dst = edge_index[1]
    g_lo = src * 2
    gidx = jnp.stack([g_lo, g_lo + 1]).reshape(NC, NS, NG, K, CHUNK)
    didx = dst.reshape(NS, NG, K, CHUNK)
    xr = x.reshape(2 * N, HALF)
    zeros = jnp.zeros((NP, HALF), jnp.float32)

    agg_lo, agg_hi = _sc_agg(xr, gidx, didx, zeros)

    inv = 1.0 / jnp.sqrt(jnp.float32(1.0) + BN_EPS)
    s1 = g1 * inv
    W1f = W1 * s1[None, :]
    b1f = (b1 * s1 + be1)[None, :]
    s2 = g2 * inv
    W2f = W2 * s2[None, :]
    b2f = (b2 * s2 + be2)[None, :]
    s0 = jnp.broadcast_to(1.0 + eps, (1, D)).astype(jnp.float32)

    out = pl.pallas_call(
        _tc_mlp_body,
        grid=(N // RB,),
        in_specs=[
            pl.BlockSpec((1, D), lambda i: (0, 0)),
            pl.BlockSpec((RB, D), lambda i: (i, 0)),
            pl.BlockSpec((RB, HALF), lambda i: (i, 0)),
            pl.BlockSpec((RB, HALF), lambda i: (i, 0)),
            pl.BlockSpec((D, D), lambda i: (0, 0)),
            pl.BlockSpec((1, D), lambda i: (0, 0)),
            pl.BlockSpec((D, D), lambda i: (0, 0)),
            pl.BlockSpec((1, D), lambda i: (0, 0)),
        ],
        out_specs=pl.BlockSpec((RB, D), lambda i: (i, 0)),
        out_shape=jax.ShapeDtypeStruct((N, D), jnp.float32),
    )(s0, x, agg_lo, agg_hi, W1f, b1f, W2f, b2f)
    return out


# speedup vs baseline: 6.3106x; 1.0571x over previous
import functools

import jax
import jax.numpy as jnp
from jax import lax
from jax.experimental import pallas as pl
from jax.experimental.pallas import tpu as pltpu
from jax.experimental.pallas import tpu_sc as plsc

N = 10000
E = 160000
D = 256
HALF = 128
BN_EPS = 1e-5

NC = 2
NS = 16
EPT = E // NS
CHUNK = 40
NCH = EPT // CHUNK
NP = 10240
RPT = NP // NS
K = 5
NG = NCH // K

_sc_mesh = plsc.VectorSubcoreMesh(core_axis_name="c", subcore_axis_name="s")


@functools.partial(
    pl.kernel,
    mesh=_sc_mesh,
    out_type=[
        jax.ShapeDtypeStruct((NP, HALF), jnp.float32),
        jax.ShapeDtypeStruct((NP, HALF), jnp.float32),
    ],
    scratch_types=[
        pltpu.VMEM((2, K, CHUNK), jnp.int32),
        pltpu.VMEM((2, K, CHUNK), jnp.int32),
        pltpu.VMEM((K, CHUNK, HALF), jnp.float32),
        pltpu.VMEM_SHARED((NP, HALF), jnp.float32),
        pltpu.SemaphoreType.DMA,
        pltpu.SemaphoreType.DMA,
        pltpu.SemaphoreType.DMA,
        pltpu.SemaphoreType.DMA,
        pltpu.SemaphoreType.DMA,
        pltpu.SemaphoreType.DMA,
        pltpu.SemaphoreType.DMA,
        pltpu.SemaphoreType.DMA,
        pltpu.SemaphoreType.DMA,
        pltpu.SemaphoreType.DMA,
        pltpu.SemaphoreType.DMA,
        pltpu.SemaphoreType.DMA,
    ],
)
def _sc_agg(xr_hbm, gidx_hbm, didx_hbm, zeros_hbm, out_lo, out_hi,
            gidx_v, didx_v, rows_v, agg_sh,
            g0s, g1s, g2s, g3s, g4s, c0s, c1s, c2s, c3s, c4s, sig, sid):
    c = lax.axis_index("c")
    s = lax.axis_index("s")
    gsems = (g0s, g1s, g2s, g3s, g4s)
    ssems = (c0s, c1s, c2s, c3s, c4s)
    pltpu.sync_copy(gidx_hbm.at[c, s, 0], gidx_v.at[0])
    pltpu.sync_copy(didx_hbm.at[s, 0], didx_v.at[0])
    pltpu.sync_copy(zeros_hbm.at[pl.ds(s * RPT, RPT)],
                    agg_sh.at[pl.ds(s * RPT, RPT)])
    plsc.subcore_barrier()

    def scat_wait(k):
        pltpu.make_async_copy(rows_v.at[k],
                              agg_sh.at[didx_v.at[0, k]], ssems[k]).wait()

    def group(j, p):
        @pl.when(j > 0)
        def _():
            for k in range(K):
                scat_wait(k)
        gh = []
        for k in range(K):
            gh.append(pltpu.async_copy(
                xr_hbm.at[gidx_v.at[p, k]], rows_v.at[k], gsems[k]))
        jn = jnp.minimum(j + 1, NG - 1)
        hg = pltpu.async_copy(gidx_hbm.at[c, s, jn], gidx_v.at[1 - p], sig)
        hd = pltpu.async_copy(didx_hbm.at[s, jn], didx_v.at[1 - p], sid)
        for k in range(K):
            gh[k].wait()
            pltpu.async_copy(rows_v.at[k], agg_sh.at[didx_v.at[p, k]],
                             ssems[k], add=True)
        hg.wait()
        hd.wait()

    def body(m, carry):
        group(2 * m, 0)
        group(2 * m + 1, 1)
        return carry

    lax.fori_loop(0, NG // 2, body, 0)
    for k in range(K):
        scat_wait(k)
    plsc.subcore_barrier()

    @pl.when(c == 0)
    def _():
        pltpu.sync_copy(agg_sh.at[pl.ds(s * RPT, RPT)],
                        out_lo.at[pl.ds(s * RPT, RPT)])

    @pl.when(c == 1)
    def _():
        pltpu.sync_copy(agg_sh.at[pl.ds(s * RPT, RPT)],
                        out_hi.at[pl.ds(s * RPT, RPT)])


RB = 2000


def _tc_mlp_body(s0_ref, x_ref, alo_ref, ahi_ref, w1_ref, b1_ref,
                 w2_ref, b2_ref, o_ref):
    x = x_ref[...]
    agg = jnp.concatenate([alo_ref[...], ahi_ref[...]], axis=1)
    h = s0_ref[...] * x + agg
    h = jnp.dot(h, w1_ref[...], preferred_element_type=jnp.float32)
    h = jnp.maximum(h + b1_ref[...], 0.0)
    h = jnp.dot(h, w2_ref[...], preferred_element_type=jnp.float32)
    h = jnp.maximum(h + b2_ref[...], 0.0)
    o_ref[...] = h + x


def kernel(x, edge_index, eps, W1, b1, g1, be1, W2, b2, g2, be2):
    src = edge_index[0]
    dst = edge_index[1]
    g_lo = src * 2
    gidx = jnp.stack([g_lo, g_lo + 1]).reshape(NC, NS, NG, K, CHUNK)
    didx = dst.reshape(NS, NG, K, CHUNK)
    xr = x.reshape(2 * N, HALF)
    zeros = jnp.zeros((NP, HALF), jnp.float32)

    agg_lo, agg_hi = _sc_agg(xr, gidx, didx, zeros)

    inv = 1.0 / jnp.sqrt(jnp.float32(1.0) + BN_EPS)
    s1 = g1 * inv
    W1f = W1 * s1[None, :]
    b1f = (b1 * s1 + be1)[None, :]
    s2 = g2 * inv
    W2f = W2 * s2[None, :]
    b2f = (b2 * s2 + be2)[None, :]
    s0 = jnp.broadcast_to(1.0 + eps, (1, D)).astype(jnp.float32)

    out = pl.pallas_call(
        _tc_mlp_body,
        grid=(N // RB,),
        in_specs=[
            pl.BlockSpec((1, D), lambda i: (0, 0)),
            pl.BlockSpec((RB, D), lambda i: (i, 0)),
            pl.BlockSpec((RB, HALF), lambda i: (i, 0)),
            pl.BlockSpec((RB, HALF), lambda i: (i, 0)),
            pl.BlockSpec((D, D), lambda i: (0, 0)),
            pl.BlockSpec((1, D), lambda i: (0, 0)),
            pl.BlockSpec((D, D), lambda i: (0, 0)),
            pl.BlockSpec((1, D), lambda i: (0, 0)),
        ],
        out_specs=pl.BlockSpec((RB, D), lambda i: (i, 0)),
        out_shape=jax.ShapeDtypeStruct((N, D), jnp.float32),
    )(s0, x, agg_lo, agg_hi, W1f, b1f, W2f, b2f)
    return out
